# Initial kernel scaffold; baseline (speedup 1.0000x reference)
import jax
import jax.numpy as jnp
from jax.experimental import pallas as pl


def kernel(pred_boxes, gt_boxes, gt_labels):
    raise NotImplementedError("write your pallas kernel here")


# speedup vs baseline: 1.3349x; 1.3349x over previous
import jax
import jax.numpy as jnp
from jax import lax
from jax.experimental import pallas as pl
from jax.experimental.pallas import tpu as pltpu

B, C, H, W, N = 16, 80, 100, 100, 32
LAMBDA_L1, LAMBDA_GIOU, POS_IOU_THR = 1.0, 2.0, 0.3
R_LO, R_HI = 2, 1


def _loss_kernel(labels_ref, pred_ref, gt_ref, out_ref):
    b = pl.program_id(0)
    n = pl.program_id(1)

    x0 = pred_ref[0, 0, 0]
    y0 = pred_ref[0, 0, 1]
    x1 = pred_ref[0, 0, 2]
    y1 = pred_ref[0, 0, 3]

    base = (b * N + n) * 4
    gx0 = gt_ref[base + 0]
    gy0 = gt_ref[base + 1]
    gx1 = gt_ref[base + 2]
    gy1 = gt_ref[base + 3]

    area_a = (x1 - x0) * (y1 - y0)
    area_b = (gx1 - gx0) * (gy1 - gy0)
    ltx = jnp.maximum(x0, gx0)
    lty = jnp.maximum(y0, gy0)
    rbx = jnp.minimum(x1, gx1)
    rby = jnp.minimum(y1, gy1)
    iw = jnp.maximum(rbx - ltx, 0.0)
    ih = jnp.maximum(rby - lty, 0.0)
    inter = iw * ih
    union = area_a + area_b - inter
    iou = inter / union
    cx = jnp.minimum(x0, gx0)
    cy = jnp.minimum(y0, gy0)
    dx = jnp.maximum(x1, gx1)
    dy = jnp.maximum(y1, gy1)
    cw = jnp.maximum(dx - cx, 0.0)
    ch = jnp.maximum(dy - cy, 0.0)
    area_c = cw * ch
    g = iou - (area_c - union) / area_c

    m1 = jnp.max(g, axis=1, keepdims=True)
    m = jnp.max(m1, axis=0, keepdims=True)
    rows_f = lax.broadcasted_iota(jnp.int32, (H, W), 0).astype(jnp.float32)
    cols_f = lax.broadcasted_iota(jnp.int32, (H, W), 1).astype(jnp.float32)
    key = rows_f * 128.0 + cols_f
    cand = jnp.where(g == m, key, 3.4e38)
    k1 = jnp.min(cand, axis=1, keepdims=True)
    kmin = jnp.min(k1, axis=0, keepdims=True)
    mi = jnp.floor(kmin * (1.0 / 128.0))
    mj = kmin - mi * 128.0

    rmask = (rows_f >= mi - float(R_LO)) & (rows_f <= jnp.minimum(mi + float(R_HI), float(H - 1)))
    cmask = (cols_f >= mj - float(R_LO)) & (cols_f <= jnp.minimum(mj + float(R_HI), float(W - 1)))
    mask = jnp.where(rmask & cmask, 1.0, 0.0)

    l1 = (jnp.abs(x0 - gx0) + jnp.abs(y0 - gy0)
          + jnp.abs(x1 - gx1) + jnp.abs(y1 - gy1)) * 0.25

    s_l1 = jnp.sum(l1 * mask, axis=1, keepdims=True)
    s_g = jnp.sum((1.0 - g) * mask, axis=1, keepdims=True)
    s_c = jnp.sum(mask, axis=1, keepdims=True)
    s_l1 = jnp.sum(s_l1, axis=0, keepdims=True)
    s_g = jnp.sum(s_g, axis=0, keepdims=True)
    s_c = jnp.sum(s_c, axis=0, keepdims=True)

    valid = jnp.where(m > POS_IOU_THR, 1.0, 0.0)

    lane = lax.broadcasted_iota(jnp.int32, (1, 128), 1)
    vals = (jnp.where(lane == 0, 1.0, 0.0) * s_l1
            + jnp.where(lane == 1, 1.0, 0.0) * s_g
            + jnp.where(lane == 2, 1.0, 0.0) * s_c) * valid
    out_ref[...] = vals.reshape(1, 1, 128)


def kernel(pred_boxes, gt_boxes, gt_labels):
    pred_t = jnp.moveaxis(pred_boxes, -1, 2)
    gt_flat = gt_boxes.reshape(-1)
    labels = gt_labels.astype(jnp.int32)

    out = pl.pallas_call(
        _loss_kernel,
        grid_spec=pltpu.PrefetchScalarGridSpec(
            num_scalar_prefetch=1,
            grid=(B, N),
            in_specs=[
                pl.BlockSpec((1, 1, 4, H, W),
                             lambda b, n, labels: (b, labels[b, n], 0, 0, 0)),
                pl.BlockSpec(memory_space=pltpu.SMEM),
            ],
            out_specs=pl.BlockSpec((1, 1, 128),
                                   lambda b, n, labels: (b * N + n, 0, 0)),
        ),
        out_shape=jax.ShapeDtypeStruct((B * N, 1, 128), jnp.float32),
        compiler_params=pltpu.CompilerParams(
            dimension_semantics=("parallel", "arbitrary"),
        ),
        name="gtclass_loc_loss",
    )(labels, pred_t, gt_flat)

    part = out[:, 0, :]
    l1_sum = jnp.sum(part[:, 0])
    g_sum = jnp.sum(part[:, 1])
    n_pos = jnp.sum(part[:, 2])
    denom = jnp.maximum(n_pos, 1.0)
    return LAMBDA_L1 * (l1_sum / denom) + LAMBDA_GIOU * (g_sum / denom)
